# Initial kernel scaffold; baseline (speedup 1.0000x reference)
import jax
import jax.numpy as jnp
from jax.experimental import pallas as pl


def kernel(sentence_lists, glove_weight):
    raise NotImplementedError("write your pallas kernel here")


# speedup vs baseline: 2.9719x; 2.9719x over previous
import functools

import jax
import jax.numpy as jnp
from jax import lax
from jax.experimental import pallas as pl
from jax.experimental.pallas import tpu as pltpu
from jax.experimental.pallas import tpu_sc as plsc

_DIM = 128
_B = 4096 * 50
_NC = 2
_NS = 16
_NW = _NC * _NS
_BPW = _B // _NW
_CHUNK = 128
_NCHUNK = _BPW // _CHUNK


def _make_emb():
    mesh = plsc.VectorSubcoreMesh(core_axis_name="c", subcore_axis_name="s")

    @functools.partial(
        pl.kernel,
        mesh=mesh,
        out_type=jax.ShapeDtypeStruct((_B, _DIM), jnp.float32),
        scratch_types=[
            pltpu.VMEM((_NCHUNK, _CHUNK), jnp.int32),
            pltpu.VMEM((_CHUNK, _DIM), jnp.float32),
            pltpu.SemaphoreType.DMA,
            pltpu.SemaphoreType.DMA,
        ],
    )
    def emb(idx_hbm, table_hbm, out_hbm, idx_v, rows_v, gsem, ssem):
        wid = lax.axis_index("s") * _NC + lax.axis_index("c")
        base = wid * _BPW
        pltpu.sync_copy(idx_hbm.at[wid], idx_v)

        def body(c, carry):
            pltpu.async_copy(table_hbm.at[idx_v.at[c]], rows_v, gsem).wait()
            pltpu.async_copy(
                rows_v, out_hbm.at[pl.ds(base + c * _CHUNK, _CHUNK)], ssem
            ).wait()
            return carry

        lax.fori_loop(0, _NCHUNK, body, 0)

    return emb


_EMB = _make_emb()


@jax.jit
def kernel(sentence_lists, glove_weight):
    idx = sentence_lists.reshape(_NW, _NCHUNK, _CHUNK).astype(jnp.int32)
    out = _EMB(idx, glove_weight)
    return out.reshape(sentence_lists.shape + (_DIM,))


# speedup vs baseline: 3.3388x; 1.1235x over previous
import functools

import jax
import jax.numpy as jnp
from jax import lax
from jax.experimental import pallas as pl
from jax.experimental.pallas import tpu as pltpu
from jax.experimental.pallas import tpu_sc as plsc

_DIM = 128
_B = 4096 * 50
_NC = 2
_NS = 16
_NW = _NC * _NS
_BPW = _B // _NW
_CHUNK = 128
_NCHUNK = _BPW // _CHUNK
_NBUF = 5
_S = 2
_NGROUP = _NCHUNK // _NBUF


def _make_emb():
    mesh = plsc.VectorSubcoreMesh(core_axis_name="c", subcore_axis_name="s")

    @functools.partial(
        pl.kernel,
        mesh=mesh,
        out_type=jax.ShapeDtypeStruct((_B, _DIM), jnp.float32),
        scratch_types=[
            pltpu.VMEM((_NCHUNK, _CHUNK), jnp.int32),
            pltpu.VMEM((_NBUF, _CHUNK, _DIM), jnp.float32),
            pltpu.SemaphoreType.DMA((_NBUF,)),
            pltpu.SemaphoreType.DMA((_NBUF,)),
        ],
    )
    def emb(idx_hbm, table_hbm, out_hbm, idx_v, rows_v, gsem, ssem):
        wid = lax.axis_index("s") * _NC + lax.axis_index("c")
        base = wid * _BPW
        pltpu.sync_copy(idx_hbm.at[wid], idx_v)

        def fire_gather(c, b):
            pltpu.async_copy(table_hbm.at[idx_v.at[c]], rows_v.at[b], gsem.at[b])

        def wait_gather(c, b):
            pltpu.make_async_copy(
                table_hbm.at[idx_v.at[c]], rows_v.at[b], gsem.at[b]
            ).wait()

        def out_slice(c):
            return out_hbm.at[pl.ds(base + c * _CHUNK, _CHUNK)]

        def fire_store(c, b):
            pltpu.async_copy(rows_v.at[b], out_slice(c), ssem.at[b])

        def wait_store(c, b):
            pltpu.make_async_copy(rows_v.at[b], out_slice(c), ssem.at[b]).wait()

        for c in range(_NBUF - _S):
            fire_gather(c, c % _NBUF)

        for b in range(_NBUF):
            c = b
            if c - _S >= 0:
                wait_store(c - _S, (c - _S) % _NBUF)
            fire_gather(c + _NBUF - _S, (c - _S) % _NBUF)
            wait_gather(c, b)
            fire_store(c, b)

        def body(g, carry):
            for b in range(_NBUF):
                c = g * _NBUF + b
                pb = (b - _S) % _NBUF
                wait_store(c - _S, pb)
                fire_gather(c + _NBUF - _S, pb)
                wait_gather(c, b)
                fire_store(c, b)
            return carry

        lax.fori_loop(1, _NGROUP - 1, body, 0)

        for b in range(_NBUF):
            c = (_NGROUP - 1) * _NBUF + b
            pb = (b - _S) % _NBUF
            wait_store(c - _S, pb)
            if c + _NBUF - _S < _NCHUNK:
                fire_gather(c + _NBUF - _S, pb)
            wait_gather(c, b)
            fire_store(c, b)
        for c in range(_NCHUNK - _S, _NCHUNK):
            wait_store(c, c % _NBUF)

    return emb


_EMB = _make_emb()


@jax.jit
def kernel(sentence_lists, glove_weight):
    idx = sentence_lists.reshape(_NW, _NCHUNK, _CHUNK).astype(jnp.int32)
    out = _EMB(idx, glove_weight)
    return out.reshape(sentence_lists.shape + (_DIM,))
